# baseline (speedup 1.0000x reference)
import functools
import jax
import jax.numpy as jnp
from jax.experimental import pallas as pl
from jax.experimental.pallas import tpu as pltpu

LN_EPS = 1e-5
PACK = 4
CHUNK = 32
TILE_E = 8192
CORE_SPLIT = 2


def _accumulate_kernel(x4_ref, ea_ref, ids_ref, dstr_ref, m_ref,
                       p_ref, r_ref, o_ref, gid_ref, q_ref, *, d_edge):
    t = pl.program_id(1)

    @pl.when(t == 0)
    def _init():
        o_ref[...] = jnp.zeros_like(o_ref)
        gid_ref[...] = jax.lax.broadcasted_iota(jnp.int32, gid_ref.shape, 1)
        q_ref[...] = jax.lax.broadcasted_iota(jnp.int32, q_ref.shape, 1) >> 5

    tile_e = ea_ref.shape[0]
    n4 = x4_ref.shape[0]

    src_col = ids_ref[:, 0:1]
    dst_col = ids_ref[:, 1:2]
    oh_src = ((src_col >> 2) == gid_ref[...]).astype(jnp.bfloat16)
    gq = jnp.dot(oh_src, x4_ref[...],
                 preferred_element_type=jnp.float32)

    gsel = jnp.where((src_col & 3) == q_ref[...], gq, 0.0
                     ).astype(jnp.bfloat16)
    g_rep = jnp.dot(gsel, r_ref[...], preferred_element_type=jnp.float32)

    ea_exp = jnp.dot(ea_ref[...], p_ref[...],
                     preferred_element_type=jnp.float32)
    u = (ea_exp * g_rep).astype(jnp.bfloat16)
    msg_rep = jnp.dot(u, m_ref[...],
                      preferred_element_type=jnp.float32)

    msg = jnp.where((dst_col & 3) == q_ref[...], msg_rep, 0.0
                    ).astype(jnp.bfloat16)
    sid = jax.lax.broadcasted_iota(jnp.int32, (n4, tile_e), 0)
    oh_dst = (sid == (dstr_ref[...] >> 2)).astype(jnp.bfloat16)
    o_ref[...] += jnp.dot(oh_dst, msg, preferred_element_type=jnp.float32)


def _finalize_kernel(part_ref, ln_ref, b_ref, o_ref, *, core_split, n4, d_out_real):
    agg = part_ref[0:n4, :]
    for c in range(1, core_split):
        agg = agg + part_ref[c * n4:(c + 1) * n4, :]

    lane = jax.lax.broadcasted_iota(jnp.int32, agg.shape, 1)
    mask = (lane % CHUNK) < d_out_real

    def block_sum(v):
        return jnp.dot(v, b_ref[...], preferred_element_type=jnp.float32)

    aggm = jnp.where(mask, agg, 0.0)
    mu = block_sum(aggm) / d_out_real
    diff = agg - mu
    diff_m = jnp.where(mask, diff, 0.0)
    var = block_sum(diff_m * diff_m) / d_out_real
    h = diff * jax.lax.rsqrt(var + LN_EPS)
    o_ref[...] = h * ln_ref[0:1, :] + ln_ref[1:2, :]


def kernel(node_features, edge_index, edge_attr, node_pos, C, W1, W2, gamma, beta):
    del node_pos
    N, d_in = node_features.shape
    E, d_edge = edge_attr.shape
    d_out = W2.shape[1]
    assert d_in <= CHUNK and d_out <= CHUNK and d_edge == PACK

    n_pad = ((N + PACK * 8 - 1) // (PACK * 8)) * (PACK * 8)
    n4 = n_pad // PACK
    x32 = jnp.zeros((n_pad, CHUNK), jnp.float32)
    x32 = x32.at[:N, :d_in].set(node_features.astype(jnp.float32))
    x4 = x32.reshape(n4, PACK * CHUNK).astype(jnp.bfloat16)

    M = jnp.einsum('ijk,kh,ho->ijo', C, W1, W2)
    m32 = jnp.zeros((d_edge, CHUNK, PACK, CHUNK), jnp.float32)
    m32 = m32.at[:, :d_in, :, :d_out].set(
        jnp.transpose(M, (1, 0, 2))[:, :, None, :])
    m32 = m32.reshape(d_edge * CHUNK, PACK * CHUNK).astype(jnp.bfloat16)

    p_exp = jnp.repeat(jnp.eye(d_edge, dtype=jnp.float32), CHUNK,
                       axis=1).astype(jnp.bfloat16)

    ln = jnp.zeros((2, CHUNK), jnp.float32)
    ln = ln.at[0, :d_out].set(gamma.astype(jnp.float32))
    ln = ln.at[1, :d_out].set(beta.astype(jnp.float32))
    ln = jnp.tile(ln, (1, PACK))

    blk = jnp.arange(PACK * CHUNK, dtype=jnp.int32) // CHUNK
    b_sum = (blk[:, None] == blk[None, :]).astype(jnp.float32)

    lane_id = jnp.arange(PACK * CHUNK, dtype=jnp.int32)
    r_rep = ((lane_id[:, None] % CHUNK) == (lane_id[None, :] % CHUNK)
             ).astype(jnp.bfloat16)

    chunk_e = CORE_SPLIT * TILE_E
    e_pad = ((E + chunk_e - 1) // chunk_e) * chunk_e
    steps = e_pad // chunk_e
    src_i = edge_index[0].astype(jnp.int32)
    dst_i = edge_index[1].astype(jnp.int32)
    ea = jnp.zeros((e_pad, d_edge), jnp.float32).at[:E].set(
        edge_attr.astype(jnp.float32)).astype(jnp.bfloat16)
    ids = jnp.zeros((e_pad, 2), jnp.int32)
    ids = ids.at[:E, 0].set(src_i).at[:E, 1].set(dst_i)
    dstr = jnp.zeros((1, e_pad), jnp.int32).at[0, :E].set(dst_i)

    partial = pl.pallas_call(
        functools.partial(_accumulate_kernel, d_edge=d_edge),
        out_shape=jax.ShapeDtypeStruct((CORE_SPLIT * n4, PACK * CHUNK), jnp.float32),
        grid=(CORE_SPLIT, steps),
        in_specs=[
            pl.BlockSpec((n4, PACK * CHUNK), lambda c, t: (0, 0)),
            pl.BlockSpec((TILE_E, d_edge), lambda c, t: (c * steps + t, 0)),
            pl.BlockSpec((TILE_E, 2), lambda c, t: (c * steps + t, 0)),
            pl.BlockSpec((1, TILE_E), lambda c, t: (0, c * steps + t)),
            pl.BlockSpec((d_edge * CHUNK, PACK * CHUNK), lambda c, t: (0, 0)),
            pl.BlockSpec((d_edge, PACK * CHUNK), lambda c, t: (0, 0)),
            pl.BlockSpec((PACK * CHUNK, PACK * CHUNK), lambda c, t: (0, 0)),
        ],
        out_specs=pl.BlockSpec((n4, PACK * CHUNK), lambda c, t: (c, 0)),
        scratch_shapes=[
            pltpu.VMEM((TILE_E, n4), jnp.int32),
            pltpu.VMEM((TILE_E, PACK * CHUNK), jnp.int32),
        ],
        compiler_params=pltpu.CompilerParams(
            dimension_semantics=("parallel", "arbitrary"),
            vmem_limit_bytes=48 * 1024 * 1024,
        ),
    )(x4, ea, ids, dstr, m32, p_exp, r_rep)

    out_p = pl.pallas_call(
        functools.partial(_finalize_kernel, core_split=CORE_SPLIT,
                          n4=n4, d_out_real=d_out),
        out_shape=jax.ShapeDtypeStruct((n4, PACK * CHUNK), jnp.float32),
        grid=(1,),
        in_specs=[
            pl.BlockSpec((CORE_SPLIT * n4, PACK * CHUNK), lambda i: (0, 0)),
            pl.BlockSpec((2, PACK * CHUNK), lambda i: (0, 0)),
            pl.BlockSpec((PACK * CHUNK, PACK * CHUNK), lambda i: (0, 0)),
        ],
        out_specs=pl.BlockSpec((n4, PACK * CHUNK), lambda i: (0, 0)),
    )(partial, ln, b_sum)

    return out_p.reshape(n_pad, CHUNK)[:N, :d_out]


# speedup vs baseline: 2.3773x; 2.3773x over previous
import functools
import jax
import jax.numpy as jnp
from jax.experimental import pallas as pl
from jax.experimental.pallas import tpu as pltpu

LN_EPS = 1e-5
PACK = 4
CHUNK = 32
TILE_E = 8192
CORE_SPLIT = 2


def _accumulate_kernel(x4_ref, ea_ref, src_ref, dstc_ref, dstr_ref, m_ref,
                       p_ref, r_ref, o_ref, gid_ref, q_ref, *, d_edge):
    t = pl.program_id(1)

    @pl.when(t == 0)
    def _init():
        o_ref[...] = jnp.zeros_like(o_ref)
        gid_ref[...] = jax.lax.broadcasted_iota(jnp.int32, gid_ref.shape, 1)
        q_ref[...] = jax.lax.broadcasted_iota(jnp.int32, q_ref.shape, 1) >> 5

    tile_e = ea_ref.shape[0]
    n4 = x4_ref.shape[0]

    oh_src = ((src_ref[...] >> 2) == gid_ref[...]).astype(jnp.bfloat16)
    gq = jnp.dot(oh_src, x4_ref[...],
                 preferred_element_type=jnp.float32)

    gsel = jnp.where((src_ref[...] & 3) == q_ref[...], gq, 0.0
                     ).astype(jnp.bfloat16)
    g_rep = jnp.dot(gsel, r_ref[...], preferred_element_type=jnp.float32)

    ea_exp = jnp.dot(ea_ref[...], p_ref[...],
                     preferred_element_type=jnp.float32)
    u = (ea_exp * g_rep).astype(jnp.bfloat16)
    msg_rep = jnp.dot(u, m_ref[...],
                      preferred_element_type=jnp.float32)

    msg = jnp.where((dstc_ref[...] & 3) == q_ref[...], msg_rep, 0.0
                    ).astype(jnp.bfloat16)
    sid = jax.lax.broadcasted_iota(jnp.int32, (n4, tile_e), 0)
    oh_dst = (sid == (dstr_ref[...] >> 2)).astype(jnp.bfloat16)
    o_ref[...] += jnp.dot(oh_dst, msg, preferred_element_type=jnp.float32)


def _finalize_kernel(part_ref, ln_ref, b_ref, o_ref, *, core_split, n4, d_out_real):
    agg = part_ref[0:n4, :]
    for c in range(1, core_split):
        agg = agg + part_ref[c * n4:(c + 1) * n4, :]

    lane = jax.lax.broadcasted_iota(jnp.int32, agg.shape, 1)
    mask = (lane % CHUNK) < d_out_real

    def block_sum(v):
        return jnp.dot(v, b_ref[...], preferred_element_type=jnp.float32)

    aggm = jnp.where(mask, agg, 0.0)
    mu = block_sum(aggm) / d_out_real
    diff = agg - mu
    diff_m = jnp.where(mask, diff, 0.0)
    var = block_sum(diff_m * diff_m) / d_out_real
    h = diff * jax.lax.rsqrt(var + LN_EPS)
    o_ref[...] = h * ln_ref[0:1, :] + ln_ref[1:2, :]


def kernel(node_features, edge_index, edge_attr, node_pos, C, W1, W2, gamma, beta):
    del node_pos
    N, d_in = node_features.shape
    E, d_edge = edge_attr.shape
    d_out = W2.shape[1]
    assert d_in <= CHUNK and d_out <= CHUNK and d_edge == PACK

    n_pad = ((N + PACK * 8 - 1) // (PACK * 8)) * (PACK * 8)
    n4 = n_pad // PACK
    x32 = jnp.zeros((n_pad, CHUNK), jnp.float32)
    x32 = x32.at[:N, :d_in].set(node_features.astype(jnp.float32))
    x4 = x32.reshape(n4, PACK * CHUNK).astype(jnp.bfloat16)

    M = jnp.einsum('ijk,kh,ho->ijo', C, W1, W2)
    m32 = jnp.zeros((d_edge, CHUNK, PACK, CHUNK), jnp.float32)
    m32 = m32.at[:, :d_in, :, :d_out].set(
        jnp.transpose(M, (1, 0, 2))[:, :, None, :])
    m32 = m32.reshape(d_edge * CHUNK, PACK * CHUNK).astype(jnp.bfloat16)

    p_exp = jnp.repeat(jnp.eye(d_edge, dtype=jnp.float32), CHUNK,
                       axis=1).astype(jnp.bfloat16)

    ln = jnp.zeros((2, CHUNK), jnp.float32)
    ln = ln.at[0, :d_out].set(gamma.astype(jnp.float32))
    ln = ln.at[1, :d_out].set(beta.astype(jnp.float32))
    ln = jnp.tile(ln, (1, PACK))

    blk = jnp.arange(PACK * CHUNK, dtype=jnp.int32) // CHUNK
    b_sum = (blk[:, None] == blk[None, :]).astype(jnp.float32)

    lane_id = jnp.arange(PACK * CHUNK, dtype=jnp.int32)
    r_rep = ((lane_id[:, None] % CHUNK) == (lane_id[None, :] % CHUNK)
             ).astype(jnp.bfloat16)

    chunk_e = CORE_SPLIT * TILE_E
    e_pad = ((E + chunk_e - 1) // chunk_e) * chunk_e
    steps = e_pad // chunk_e
    src_i = edge_index[0].astype(jnp.int32)
    dst_i = edge_index[1].astype(jnp.int32)
    ea = jnp.zeros((e_pad, d_edge), jnp.float32).at[:E].set(
        edge_attr.astype(jnp.float32)).astype(jnp.bfloat16)
    src = jnp.zeros((e_pad, 1), jnp.int32).at[:E, 0].set(src_i)
    dstc = jnp.zeros((e_pad, 1), jnp.int32).at[:E, 0].set(dst_i)
    dstr = jnp.zeros((1, e_pad), jnp.int32).at[0, :E].set(dst_i)

    partial = pl.pallas_call(
        functools.partial(_accumulate_kernel, d_edge=d_edge),
        out_shape=jax.ShapeDtypeStruct((CORE_SPLIT * n4, PACK * CHUNK), jnp.float32),
        grid=(CORE_SPLIT, steps),
        in_specs=[
            pl.BlockSpec((n4, PACK * CHUNK), lambda c, t: (0, 0)),
            pl.BlockSpec((TILE_E, d_edge), lambda c, t: (c * steps + t, 0)),
            pl.BlockSpec((TILE_E, 1), lambda c, t: (c * steps + t, 0)),
            pl.BlockSpec((TILE_E, 1), lambda c, t: (c * steps + t, 0)),
            pl.BlockSpec((1, TILE_E), lambda c, t: (0, c * steps + t)),
            pl.BlockSpec((d_edge * CHUNK, PACK * CHUNK), lambda c, t: (0, 0)),
            pl.BlockSpec((d_edge, PACK * CHUNK), lambda c, t: (0, 0)),
            pl.BlockSpec((PACK * CHUNK, PACK * CHUNK), lambda c, t: (0, 0)),
        ],
        out_specs=pl.BlockSpec((n4, PACK * CHUNK), lambda c, t: (c, 0)),
        scratch_shapes=[
            pltpu.VMEM((TILE_E, n4), jnp.int32),
            pltpu.VMEM((TILE_E, PACK * CHUNK), jnp.int32),
        ],
        compiler_params=pltpu.CompilerParams(
            dimension_semantics=("parallel", "arbitrary"),
            vmem_limit_bytes=48 * 1024 * 1024,
        ),
    )(x4, ea, src, dstc, dstr, m32, p_exp, r_rep)

    out_p = pl.pallas_call(
        functools.partial(_finalize_kernel, core_split=CORE_SPLIT,
                          n4=n4, d_out_real=d_out),
        out_shape=jax.ShapeDtypeStruct((n4, PACK * CHUNK), jnp.float32),
        grid=(1,),
        in_specs=[
            pl.BlockSpec((CORE_SPLIT * n4, PACK * CHUNK), lambda i: (0, 0)),
            pl.BlockSpec((2, PACK * CHUNK), lambda i: (0, 0)),
            pl.BlockSpec((PACK * CHUNK, PACK * CHUNK), lambda i: (0, 0)),
        ],
        out_specs=pl.BlockSpec((n4, PACK * CHUNK), lambda i: (0, 0)),
    )(partial, ln, b_sum)

    return out_p.reshape(n_pad, CHUNK)[:N, :d_out]
